# baseline (speedup 1.0000x reference)
import jax
import jax.numpy as jnp
from jax import lax
from jax.experimental import pallas as pl
from jax.experimental.pallas import tpu as pltpu

_TILE = 1024
_LANES = 128


def _fused_body(state_size_pad, state_size, state_ref, action_ref, values_ref,
                out_ref):
    s = state_ref[...].astype(jnp.bfloat16)
    lane = lax.broadcasted_iota(jnp.int32, (_TILE, state_size_pad), 1)
    s = jnp.where(lane < state_size, s, jnp.bfloat16(0))
    v = values_ref[...].astype(jnp.bfloat16)
    q = jnp.dot(s, v, preferred_element_type=jnp.float32)
    r = (action_ref[...] * q).reshape(_TILE // _LANES, _LANES, -1)
    out_ref[...] = jnp.sum(r, axis=2)


def kernel(state, action, values):
    batch, state_size = state.shape
    action_size = action.shape[1]
    state_size_pad = (state_size + _LANES - 1) // _LANES * _LANES
    values_pad = jnp.pad(values, ((0, state_size_pad - state_size), (0, 0)))
    sub = _TILE // _LANES
    grid = (batch // _TILE,)

    import functools
    body = functools.partial(_fused_body, state_size_pad, state_size)
    out = pl.pallas_call(
        body,
        grid=grid,
        in_specs=[
            pl.BlockSpec((_TILE, state_size_pad), lambda i: (i, 0)),
            pl.BlockSpec((_TILE, action_size), lambda i: (i, 0)),
            pl.BlockSpec((state_size_pad, action_size), lambda i: (0, 0)),
        ],
        out_specs=pl.BlockSpec((sub, _LANES), lambda i: (i, 0)),
        out_shape=jax.ShapeDtypeStruct((batch // _LANES, _LANES), jnp.float32),
        compiler_params=pltpu.CompilerParams(
            dimension_semantics=("arbitrary",),
        ),
    )(state, action, values_pad)
    return out.reshape(batch, 1)
